# speedup vs baseline: 1.1335x; 1.1335x over previous
import functools

import jax
import jax.numpy as jnp
from jax.experimental import pallas as pl
from jax.experimental.pallas import tpu as pltpu

B, H, S, D = 8, 16, 2048, 128
Q = 32
BH = B * H


def _body(pos_ref, kc_ref, vc_ref, kv_ref, vv_ref, ko_ref, vo_ref):
    ko_ref[...] = kc_ref[...]
    vo_ref[...] = vc_ref[...]
    for q in range(Q):
        p = pos_ref[q]
        ko_ref[0, pl.ds(p, 1), :] = kv_ref[0, pl.ds(q, 1), :]
        vo_ref[0, pl.ds(p, 1), :] = vv_ref[0, pl.ds(q, 1), :]


@jax.jit
def kernel(k_cache, v_cache, input_pos, k_val, v_val):
    kc = k_cache.reshape(BH, S, D)
    vc = v_cache.reshape(BH, S, D)
    kv = k_val.reshape(BH, Q, D)
    vv = v_val.reshape(BH, Q, D)

    grid_spec = pltpu.PrefetchScalarGridSpec(
        num_scalar_prefetch=1,
        grid=(BH,),
        in_specs=[
            pl.BlockSpec((1, S, D), lambda i, pos: (i, 0, 0)),
            pl.BlockSpec((1, S, D), lambda i, pos: (i, 0, 0)),
            pl.BlockSpec((1, Q, D), lambda i, pos: (i, 0, 0)),
            pl.BlockSpec((1, Q, D), lambda i, pos: (i, 0, 0)),
        ],
        out_specs=[
            pl.BlockSpec((1, S, D), lambda i, pos: (i, 0, 0)),
            pl.BlockSpec((1, S, D), lambda i, pos: (i, 0, 0)),
        ],
    )
    ko, vo = pl.pallas_call(
        _body,
        grid_spec=grid_spec,
        out_shape=[
            jax.ShapeDtypeStruct((BH, S, D), jnp.float32),
            jax.ShapeDtypeStruct((BH, S, D), jnp.float32),
        ],
        compiler_params=pltpu.CompilerParams(
            dimension_semantics=("parallel",),
        ),
    )(input_pos, kc, vc, kv, vv)
    return (ko.reshape(B, H, S, D), vo.reshape(B, H, S, D))
